# baseline (speedup 1.0000x reference)
import functools

import jax
import jax.numpy as jnp
from jax import lax
from jax.experimental import pallas as pl
from jax.experimental.pallas import tpu as pltpu
from jax.experimental.pallas import tpu_sc as plsc


def _make_sc_add(S, D, B, R, xbase):
    info = plsc.get_sparse_core_info()
    NC, NS = info.num_cores, info.num_subcores
    NW = NC * NS
    rspan = S // NW
    T = rspan // R
    CD = D // 16
    assert S % NW == 0 and rspan % R == 0 and T % 2 == 0 and D % 16 == 0
    assert T * B >= 2
    mesh = plsc.VectorSubcoreMesh(core_axis_name="c", subcore_axis_name="s")

    @functools.partial(
        pl.kernel,
        mesh=mesh,
        out_type=jax.ShapeDtypeStruct((B * S, D), jnp.float32),
        scratch_types=[
            pltpu.VMEM((R, D), jnp.float32),
            pltpu.VMEM((R, D), jnp.float32),
            pltpu.VMEM((R, D), jnp.float32),
            pltpu.VMEM((R, D), jnp.float32),
            pltpu.VMEM((R, D), jnp.float32),
            pltpu.VMEM((R, D), jnp.float32),
            pltpu.SemaphoreType.DMA,
            pltpu.SemaphoreType.DMA,
            pltpu.SemaphoreType.DMA,
            pltpu.SemaphoreType.DMA,
            pltpu.SemaphoreType.DMA,
            pltpu.SemaphoreType.DMA,
        ],
    )
    def k(x_hbm, e_hbm, o_hbm, xin0, xin1, xout0, xout1, eb0, eb1,
          sxin0, sxin1, so0, so1, se0, se1):
        xin = (xin0, xin1)
        xout = (xout0, xout1)
        eb = (eb0, eb1)
        sxin = (sxin0, sxin1)
        so = (so0, so1)
        se = (se0, se1)
        wid = lax.axis_index("s") * NC + lax.axis_index("c")
        wrbase = wid * rspan

        def orow(t, b):
            return b * S + wrbase + t * R

        def erow(t):
            return wrbase + t * R

        def xload(t, b, i):
            pltpu.make_async_copy(
                x_hbm.at[pl.ds(xbase + orow(t, b), R)], xin[i], sxin[i]).start()

        def eload(t, i):
            pltpu.make_async_copy(
                e_hbm.at[pl.ds(erow(t), R)], eb[i], se[i]).start()

        eload(0, 0)
        eload(1, 1)
        xload(0, 0, 0)
        xload(1 // B, 1 % B, 1)

        def body(it, carry):
            t0 = it * 2
            for tt in range(2):
                t = t0 + tt
                pltpu.make_async_copy(
                    e_hbm.at[pl.ds(erow(t), R)], eb[tt], se[tt]).wait()
                for b in range(B):
                    i = (tt * B + b) % 2
                    pltpu.make_async_copy(
                        x_hbm.at[pl.ds(xbase + orow(t, b), R)], xin[i],
                        sxin[i]).wait()

                    def drain():
                        pltpu.make_async_copy(
                            xout[i], o_hbm.at[pl.ds(0, R)], so[i]).wait()

                    if tt * B + b < 2:
                        pl.when(t0 > 0)(drain)
                    else:
                        drain()

                    @plsc.parallel_loop(0, R * CD, unroll=8)
                    def add16(j):
                        r = j // CD
                        sl = pl.ds((j % CD) * 16, 16)
                        xout[i][r, sl] = xin[i][r, sl] + eb[tt][r, sl]

                    nt = t + (b + 2) // B
                    nb = (b + 2) % B

                    def prefetch():
                        xload(nt, nb, i)

                    if b + 2 >= B:
                        pl.when(nt < T)(prefetch)
                    else:
                        prefetch()

                    pltpu.make_async_copy(
                        xout[i], o_hbm.at[pl.ds(orow(t, b), R)], so[i]).start()

                def eprefetch():
                    eload(t + 2, tt)

                pl.when(t + 2 < T)(eprefetch)
            return carry

        lax.fori_loop(0, T // 2, body, 0)

        pltpu.make_async_copy(xout[0], o_hbm.at[pl.ds(0, R)], so[0]).wait()
        pltpu.make_async_copy(xout[1], o_hbm.at[pl.ds(0, R)], so[1]).wait()

    return k


def _tc_add_body(x_ref, e_ref, o_ref):
    o_ref[...] = x_ref[...] + e_ref[...][None]


def _tc_add(x, emb, BT, TS):
    B, S, D = x.shape
    return pl.pallas_call(
        _tc_add_body,
        grid=(S // TS, BT),
        in_specs=[
            pl.BlockSpec((1, TS, D), lambda i, j: (j, i, 0)),
            pl.BlockSpec((TS, D), lambda i, j: (i, 0)),
        ],
        out_specs=pl.BlockSpec((1, TS, D), lambda i, j: (j, i, 0)),
        out_shape=jax.ShapeDtypeStruct((BT, S, D), x.dtype),
    )(x, emb)


def kernel(x, emb):
    B, S, D = x.shape
    BT = B - 1
    BS = B - BT
    sc = _make_sc_add(S, D, BS, 16, BT * S)
    sc_out = sc(x.reshape(B * S, D), emb[:S])
    tc_out = _tc_add(x, emb[:S], BT, 2048)
    return jnp.concatenate([tc_out, sc_out.reshape(BS, S, D)], axis=0)


# speedup vs baseline: 1.6349x; 1.6349x over previous
import functools

import jax
import jax.numpy as jnp
from jax import lax
from jax.experimental import pallas as pl
from jax.experimental.pallas import tpu as pltpu
from jax.experimental.pallas import tpu_sc as plsc


def _make_sc_add(S, D, B, R, KREG):
    info = plsc.get_sparse_core_info()
    NC, NS = info.num_cores, info.num_subcores
    NW = NC * NS
    rspan = S // NW
    T = rspan // R
    NK = R * D // (16 * KREG)
    assert S % NW == 0 and rspan % R == 0 and T % 2 == 0
    assert R * D % (16 * KREG) == 0 and D % (16 * KREG) == 0
    mesh = plsc.VectorSubcoreMesh(core_axis_name="c", subcore_axis_name="s")

    @functools.partial(
        pl.kernel,
        mesh=mesh,
        out_type=jax.ShapeDtypeStruct((B * S, D), jnp.float32),
        scratch_types=(
            [pltpu.VMEM((R, D), jnp.float32) for _ in range(2 * B)]
            + [pltpu.VMEM((R, D), jnp.float32) for _ in range(B)]
            + [pltpu.VMEM((R, D), jnp.float32) for _ in range(2)]
            + [
                pltpu.SemaphoreType.DMA,
                pltpu.SemaphoreType.DMA,
                pltpu.SemaphoreType.DMA,
                pltpu.SemaphoreType.DMA,
                pltpu.SemaphoreType.DMA,
            ]
        ),
    )
    def k(x_hbm, e_hbm, o_hbm, *refs):
        xin = tuple(tuple(refs[b * 2 + p] for p in range(2)) for b in range(B))
        xout = tuple(refs[2 * B + b] for b in range(B))
        eb = (refs[3 * B], refs[3 * B + 1])
        sxl = (refs[3 * B + 2], refs[3 * B + 3])
        se = (refs[3 * B + 4], refs[3 * B + 5])
        sso = refs[3 * B + 6]
        wid = lax.axis_index("s") * NC + lax.axis_index("c")
        wrbase = wid * rspan

        def xrow(t, b):
            return b * S + wrbase + t * R

        def erow(t):
            return wrbase + t * R

        def xload(t, b, p):
            pltpu.make_async_copy(
                x_hbm.at[pl.ds(xrow(t, b), R)], xin[b][p], sxl[p]).start()

        def eload(t, p):
            pltpu.make_async_copy(
                e_hbm.at[pl.ds(erow(t), R)], eb[p], se[p]).start()

        for p in range(2):
            eload(p, p)
            for b in range(B):
                xload(p, b, p)

        def body(it, carry):
            t0 = it * 2
            for tt in range(2):
                t = t0 + tt
                pltpu.make_async_copy(
                    e_hbm.at[pl.ds(erow(t), R)], eb[tt], se[tt]).wait()
                for b in range(B):
                    pltpu.make_async_copy(
                        x_hbm.at[pl.ds(xrow(t, b), R)], xin[b][tt],
                        sxl[tt]).wait()

                def drain():
                    for b in range(B):
                        pltpu.make_async_copy(
                            xout[b], o_hbm.at[pl.ds(0, R)], sso).wait()

                if tt == 0:
                    pl.when(t0 > 0)(drain)
                else:
                    drain()

                @plsc.parallel_loop(0, NK, unroll=1)
                def addk(kk):
                    r = kk // (D // (16 * KREG))
                    c0 = (kk % (D // (16 * KREG))) * (16 * KREG)
                    evals = [
                        eb[tt][r, pl.ds(c0 + j * 16, 16)] for j in range(KREG)
                    ]
                    for b in range(B):
                        for j in range(KREG):
                            sl = pl.ds(c0 + j * 16, 16)
                            xout[b][r, sl] = xin[b][tt][r, sl] + evals[j]

                for b in range(B):
                    pltpu.make_async_copy(
                        xout[b], o_hbm.at[pl.ds(xrow(t, b), R)], sso).start()

                def prefetch():
                    eload(t + 2, tt)
                    for b in range(B):
                        xload(t + 2, b, tt)

                pl.when(t + 2 < T)(prefetch)
            return carry

        lax.fori_loop(0, T // 2, body, 0)

        for b in range(B):
            pltpu.make_async_copy(xout[b], o_hbm.at[pl.ds(0, R)], sso).wait()

    return k


def kernel(x, emb):
    B, S, D = x.shape
    k = _make_sc_add(S, D, B, 8, 8)
    out = k(x.reshape(B * S, D), emb[:S])
    return out.reshape(B, S, D)
